# baseline (speedup 1.0000x reference)
import functools

import jax
import jax.numpy as jnp
from jax import lax
from jax.experimental import pallas as pl
from jax.experimental.pallas import tpu as pltpu
from jax.experimental.pallas import tpu_sc as plsc

N_STATES_P1 = 1024
N_OBVS_P1 = 10000
N_TOTAL = 16384 * 200

_info = plsc.get_sparse_core_info()
_NC, _NS, _L = _info.num_cores, _info.num_subcores, _info.num_lanes
_NW = _NC * _NS
_PER_TILE = N_TOTAL // _NW
_CHUNK = 12800
_N_CHUNKS = _PER_TILE // _CHUNK

_mesh = plsc.VectorSubcoreMesh(core_axis_name="c", subcore_axis_name="s")


@functools.partial(
    pl.kernel,
    out_type=jax.ShapeDtypeStruct((N_TOTAL,), jnp.float32),
    mesh=_mesh,
    scratch_types=[
        pltpu.VMEM((_CHUNK,), jnp.int32),
        pltpu.VMEM((_CHUNK,), jnp.int32),
        pltpu.VMEM((_CHUNK,), jnp.int32),
        pltpu.VMEM((_CHUNK,), jnp.float32),
        pltpu.SemaphoreType.DMA,
    ],
)
def _sc_gather(table_hbm, state_hbm, obs_hbm, out_hbm, s_v, o_v, idx_v, g_v, sem):
    wid = lax.axis_index("s") * _NC + lax.axis_index("c")
    base = wid * _PER_TILE
    for c in range(_N_CHUNKS):
        off = base + c * _CHUNK
        pltpu.sync_copy(state_hbm.at[pl.ds(off, _CHUNK)], s_v)
        pltpu.sync_copy(obs_hbm.at[pl.ds(off, _CHUNK)], o_v)

        def body(i, carry):
            sl = pl.ds(i * _L, _L)
            idx_v[sl] = s_v[sl] * N_OBVS_P1 + o_v[sl]
            return carry

        lax.fori_loop(0, _CHUNK // _L, body, 0)
        pltpu.async_copy(table_hbm.at[idx_v], g_v, sem).wait()
        pltpu.sync_copy(g_v, out_hbm.at[pl.ds(off, _CHUNK)])


def kernel(state, obs, log_em):
    table = log_em.reshape(-1)
    out = _sc_gather(table, state.reshape(-1), obs.reshape(-1))
    return out.reshape(state.shape)


# speedup vs baseline: 1.1259x; 1.1259x over previous
import functools

import jax
import jax.numpy as jnp
from jax import lax
from jax.experimental import pallas as pl
from jax.experimental.pallas import tpu as pltpu
from jax.experimental.pallas import tpu_sc as plsc

N_STATES_P1 = 1024
N_OBVS_P1 = 10000
N_TOTAL = 16384 * 200

_info = plsc.get_sparse_core_info()
_NC, _NS, _L = _info.num_cores, _info.num_subcores, _info.num_lanes
_NW = _NC * _NS
_PER_TILE = N_TOTAL // _NW
_CHUNK = 12800
_N_CHUNKS = _PER_TILE // _CHUNK

_mesh = plsc.VectorSubcoreMesh(core_axis_name="c", subcore_axis_name="s")


@functools.partial(
    pl.kernel,
    out_type=jax.ShapeDtypeStruct((N_TOTAL,), jnp.float32),
    mesh=_mesh,
    scratch_types=[
        pltpu.VMEM((_CHUNK,), jnp.int32),
        pltpu.VMEM((_CHUNK,), jnp.int32),
        pltpu.VMEM((_CHUNK,), jnp.int32),
        pltpu.VMEM((_CHUNK,), jnp.int32),
        pltpu.VMEM((_CHUNK,), jnp.int32),
        pltpu.VMEM((_CHUNK,), jnp.int32),
        pltpu.VMEM((_CHUNK,), jnp.float32),
        pltpu.VMEM((_CHUNK,), jnp.float32),
        pltpu.SemaphoreType.DMA,
        pltpu.SemaphoreType.DMA,
        pltpu.SemaphoreType.DMA,
    ],
)
def _sc_gather(table_hbm, state_hbm, obs_hbm, out_hbm,
               s0, s1, s2, o0, o1, o2, g0, g1, in_sem, g_sem, wb_sem):
    s = (s0, s1, s2)
    o = (o0, o1, o2)
    g = (g0, g1)
    wid = lax.axis_index("s") * _NC + lax.axis_index("c")
    base = wid * _PER_TILE

    def start_in(c, b):
        off = base + c * _CHUNK
        h1 = pltpu.async_copy(state_hbm.at[pl.ds(off, _CHUNK)], s[b], in_sem)
        h2 = pltpu.async_copy(obs_hbm.at[pl.ds(off, _CHUNK)], o[b], in_sem)
        return (h1, h2)

    in_h = {0: start_in(0, 0)}
    g_h = {}
    wb_h = {}
    for c in range(_N_CHUNKS):
        b = c % 3
        for h in in_h.pop(c):
            h.wait()
        if c + 1 < _N_CHUNKS:
            in_h[c + 1] = start_in(c + 1, (c + 1) % 3)

        sb, ob = s[b], o[b]

        @plsc.parallel_loop(0, _CHUNK, step=_L, unroll=8)
        def _(i):
            sl = pl.ds(i, _L)
            sb[sl] = sb[sl] * N_OBVS_P1 + ob[sl]

        if c >= 1:
            g_h.pop(c - 1).wait()
            off_p = base + (c - 1) * _CHUNK
            wb_h[c - 1] = pltpu.async_copy(
                g[(c - 1) & 1], out_hbm.at[pl.ds(off_p, _CHUNK)], wb_sem)
        if c >= 2:
            wb_h.pop(c - 2).wait()
        g_h[c] = pltpu.async_copy(table_hbm.at[s[b]], g[c & 1], g_sem)

    last = _N_CHUNKS - 1
    g_h.pop(last).wait()
    wb_h[last] = pltpu.async_copy(
        g[last & 1], out_hbm.at[pl.ds(base + last * _CHUNK, _CHUNK)], wb_sem)
    wb_h.pop(last - 1).wait()
    wb_h.pop(last).wait()


def kernel(state, obs, log_em):
    table = log_em.reshape(-1)
    out = _sc_gather(table, state.reshape(-1), obs.reshape(-1))
    return out.reshape(state.shape)


# speedup vs baseline: 1.2296x; 1.0921x over previous
import functools

import jax
import jax.numpy as jnp
from jax import lax
from jax.experimental import pallas as pl
from jax.experimental.pallas import tpu as pltpu
from jax.experimental.pallas import tpu_sc as plsc

N_ROWS = 16384
ROW = 200
N_OBVS_P1 = 10000
N_TOTAL = N_ROWS * ROW

_info = plsc.get_sparse_core_info()
_NC, _NS, _L = _info.num_cores, _info.num_subcores, _info.num_lanes
_NW = _NC * _NS
_ROWS_PER_TILE = N_ROWS // _NW
_CROWS = 64
_CHUNK = _CROWS * ROW
_N_CHUNKS = _ROWS_PER_TILE // _CROWS

_CSTARTS = tuple(range(0, ROW - _L, _L)) + (ROW - _L,)

_mesh = plsc.VectorSubcoreMesh(core_axis_name="c", subcore_axis_name="s")


@functools.partial(
    pl.kernel,
    out_type=jax.ShapeDtypeStruct((N_TOTAL,), jnp.float32),
    mesh=_mesh,
    scratch_types=[
        pltpu.VMEM((_CROWS, ROW), jnp.int32),
        pltpu.VMEM((_CROWS, ROW), jnp.int32),
        pltpu.VMEM((_CROWS, ROW), jnp.int32),
        pltpu.VMEM((_CROWS, ROW), jnp.int32),
        pltpu.VMEM((_CHUNK,), jnp.int32),
        pltpu.VMEM((_CHUNK,), jnp.int32),
        pltpu.VMEM((_CHUNK,), jnp.float32),
        pltpu.VMEM((_CHUNK,), jnp.float32),
        pltpu.SemaphoreType.DMA,
        pltpu.SemaphoreType.DMA,
        pltpu.SemaphoreType.DMA,
    ],
)
def _sc_gather(table_hbm, state_hbm, obs_hbm, out_hbm,
               s0, s1, o0, o1, i0, i1, g0, g1, in_sem, g_sem, wb_sem):
    s = (s0, s1)
    o = (o0, o1)
    idx = (i0, i1)
    g = (g0, g1)
    wid = lax.axis_index("s") * _NC + lax.axis_index("c")
    row_base = wid * _ROWS_PER_TILE
    elem_base = row_base * ROW

    def start_in(c, b):
        r0 = row_base + c * _CROWS
        h1 = pltpu.async_copy(state_hbm.at[pl.ds(r0, _CROWS), :], s[b], in_sem)
        h2 = pltpu.async_copy(obs_hbm.at[pl.ds(r0, _CROWS), :], o[b], in_sem)
        return (h1, h2)

    in_h = {0: start_in(0, 0)}
    g_h = {}
    wb_h = {}
    for c in range(_N_CHUNKS):
        b = c & 1
        for h in in_h.pop(c):
            h.wait()
        if c + 1 < _N_CHUNKS:
            in_h[c + 1] = start_in(c + 1, 1 - b)

        sb, ob, ib = s[b], o[b], idx[b]

        @plsc.parallel_loop(0, _CROWS, step=1, unroll=2)
        def _(r):
            for cs in _CSTARTS:
                ib[pl.ds(r * ROW + cs, _L)] = (
                    sb[r, pl.ds(cs, _L)] * N_OBVS_P1 + ob[r, pl.ds(cs, _L)])

        if c >= 1:
            g_h.pop(c - 1).wait()
            off_p = elem_base + (c - 1) * _CHUNK
            wb_h[c - 1] = pltpu.async_copy(
                g[(c - 1) & 1], out_hbm.at[pl.ds(off_p, _CHUNK)], wb_sem)
        if c >= 2:
            wb_h.pop(c - 2).wait()
        g_h[c] = pltpu.async_copy(table_hbm.at[ib], g[b], g_sem)

    last = _N_CHUNKS - 1
    g_h.pop(last).wait()
    wb_h[last] = pltpu.async_copy(
        g[last & 1], out_hbm.at[pl.ds(elem_base + last * _CHUNK, _CHUNK)], wb_sem)
    wb_h.pop(last - 1).wait()
    wb_h.pop(last).wait()


def kernel(state, obs, log_em):
    table = log_em.reshape(-1)
    out = _sc_gather(table, state, obs)
    return out.reshape(state.shape)
